# Initial kernel scaffold; baseline (speedup 1.0000x reference)
import jax
import jax.numpy as jnp
from jax.experimental import pallas as pl


def kernel(article_sentences, article_sentences_lengths, attention, num_codes):
    raise NotImplementedError("write your pallas kernel here")


# speedup vs baseline: 27.7213x; 27.7213x over previous
import jax
import jax.numpy as jnp
from jax import lax
from jax.experimental import pallas as pl
from jax.experimental.pallas import tpu as pltpu

_B, _C, _S, _L = 8, 8, 128, 32


def _body(sent_ref, sent_t_ref, len_ref, att_ref, nc_ref,
          sa_ref, si_ref, gi_ref):
    S, L, C = _S, _L, _C
    sent = sent_ref[0]
    sent_t = sent_t_ref[0]
    att = att_ref[0]
    nc = nc_ref[0, 0, 0]

    sa = jnp.sum(att, axis=-1)
    empty = len_ref[0] == 0
    sa = jnp.where(empty, jnp.float32(0.0), sa)
    sa_ref[0] = sa

    ii = lax.broadcasted_iota(jnp.int32, (S, S), 0)
    jj = lax.broadcasted_iota(jnp.int32, (S, S), 1)
    jlt = jj < ii
    vi = sa[:, :, None]
    vj = sa[:, None, :]
    before = (vj > vi) | ((vj == vi) & jlt[None])
    rank = jnp.sum(before.astype(jnp.int32), axis=2)

    eq = jlt >= jlt
    for g in range(0, L, 3):
        col = sent[:, g:g + 1]
        row = sent_t[g:g + 1, :]
        if g + 1 < L:
            col = col * 1024 + sent[:, g + 1:g + 2]
            row = row * 1024 + sent_t[g + 1:g + 2, :]
        if g + 2 < L:
            col = col * 1024 + sent[:, g + 2:g + 3]
            row = row * 1024 + sent_t[g + 2:g + 3, :]
        eq = eq & (col == row)

    big = jnp.int32(32767)
    rank_j = rank[:, None, :]
    lead = jnp.min(jnp.where(eq[None], rank_j, big), axis=2)
    isldr = lead == rank
    gval = jnp.sum((isldr[:, None, :] & (rank_j <= lead[:, :, None]))
                   .astype(jnp.int32), axis=2) - 1

    i_row = lax.broadcasted_iota(jnp.int32, (C, S), 1)
    enc = gval * 128 + i_row
    hit = rank[:, None, :] == ii[None]
    out = jnp.sum(jnp.where(hit, enc[:, None, :], 0), axis=2)

    si_ref[0] = jnp.bitwise_and(out, 127)
    c_col = lax.broadcasted_iota(jnp.int32, (C, S), 0)
    gi_ref[0] = jnp.where(c_col < nc, out >> 7, jnp.int32(-1))


def kernel(article_sentences, article_sentences_lengths, attention, num_codes):
    B, C, S, L = _B, _C, _S, _L
    sent = article_sentences.astype(jnp.int32)
    sent_t = jnp.swapaxes(sent, 1, 2)
    lens = article_sentences_lengths.astype(jnp.int32).reshape(B, 1, S)
    nc = num_codes.astype(jnp.int32).reshape(B, 1, 1)

    grid = (B,)
    out = pl.pallas_call(
        _body,
        grid=grid,
        in_specs=[
            pl.BlockSpec((1, S, L), lambda b: (b, 0, 0)),
            pl.BlockSpec((1, L, S), lambda b: (b, 0, 0)),
            pl.BlockSpec((1, 1, S), lambda b: (b, 0, 0)),
            pl.BlockSpec((1, C, S, L), lambda b: (b, 0, 0, 0)),
            pl.BlockSpec((1, 1, 1), lambda b: (b, 0, 0)),
        ],
        out_specs=[
            pl.BlockSpec((1, C, S), lambda b: (b, 0, 0)),
            pl.BlockSpec((1, C, S), lambda b: (b, 0, 0)),
            pl.BlockSpec((1, C, S), lambda b: (b, 0, 0)),
        ],
        out_shape=[
            jax.ShapeDtypeStruct((B, C, S), jnp.float32),
            jax.ShapeDtypeStruct((B, C, S), jnp.int32),
            jax.ShapeDtypeStruct((B, C, S), jnp.int32),
        ],
        compiler_params=pltpu.CompilerParams(
            dimension_semantics=("parallel",),
        ),
    )(sent, sent_t, lens, attention, nc)
    return out[0], out[1], out[2]


# speedup vs baseline: 606.4460x; 21.8765x over previous
import jax
import jax.numpy as jnp
from jax import lax
from jax.experimental import pallas as pl
from jax.experimental.pallas import tpu as pltpu

_B, _C, _S, _L = 8, 8, 128, 32


def _rank_both(k_row, k_col, low, up):
    before = (k_row + low) > k_col
    rank_col = jnp.sum(before, axis=1, keepdims=True)
    before_t = (k_col + up) > k_row
    rank_row = jnp.sum(before_t, axis=0, keepdims=True)
    return rank_col, rank_row


def _scatter_rows(rank_col, enc_col, iota_row):
    hit_t = rank_col == iota_row
    return jnp.sum(jnp.where(hit_t, enc_col, 0), axis=0, keepdims=True)


def _body(sent_ref, sent_t_ref, len_ref, att_ref, nc_ref,
          sa_ref, si_ref, gi_ref):
    S, L, C = _S, _L, _C
    sent = sent_ref[0]
    sent_t = sent_t_ref[0]
    att = att_ref[0]
    nc = nc_ref[0, 0, 0]

    sa = jnp.sum(att, axis=-1)
    empty = len_ref[0] == 0
    sa = jnp.where(empty, jnp.float32(0.0), sa)
    sa_ref[0] = sa

    u = lax.bitcast_convert_type(sa, jnp.int32)
    k = u * 2 - jnp.int32(0x42000000)
    k_t = jnp.transpose(k)

    i0 = lax.broadcasted_iota(jnp.int32, (S, S), 0)
    i1 = lax.broadcasted_iota(jnp.int32, (S, S), 1)
    low = (i1 < i0).astype(jnp.int32)
    up = (i0 < i1).astype(jnp.int32)
    iota_row = lax.broadcasted_iota(jnp.int32, (1, S), 1)
    iota_col = lax.broadcasted_iota(jnp.int32, (S, 1), 0)

    f = jnp.concatenate([sent >> 5, sent & 31], axis=1).astype(jnp.float32)
    f_t = jnp.concatenate([sent_t >> 5, sent_t & 31], axis=0).astype(jnp.float32)
    g = jnp.dot(f, f_t, preferred_element_type=jnp.float32)
    n_col = jnp.sum(f * f, axis=1, keepdims=True)
    n_row = jnp.sum(f_t * f_t, axis=0, keepdims=True)
    eq = (g == n_col) & (g == n_row)
    n_eq = jnp.sum(eq.astype(jnp.int32))

    def finish(c, out_row):
        si_ref[0, c:c + 1] = jnp.bitwise_and(out_row, 127)
        gi = jnp.where(jnp.int32(c) < nc, out_row >> 7, jnp.int32(-1))
        gi_ref[0, c:c + 1] = gi

    @pl.when(n_eq == S)
    def _no_dups():
        for c in range(C):
            k_row = k[c:c + 1, :]
            k_col = k_t[:, c:c + 1]
            rank_col, _ = _rank_both(k_row, k_col, low, up)
            enc_col = rank_col * 128 + iota_col
            finish(c, _scatter_rows(rank_col, enc_col, iota_row))

    @pl.when(n_eq != S)
    def _dups():
        _BIG = jnp.int32(32767)
        for c in range(C):
            k_row = k[c:c + 1, :]
            k_col = k_t[:, c:c + 1]
            rank_col, rank_row = _rank_both(k_row, k_col, low, up)
            lead_col = jnp.min(jnp.where(eq, rank_row, _BIG), axis=1,
                               keepdims=True)
            lead_row = jnp.min(jnp.where(eq, rank_col, _BIG), axis=0,
                               keepdims=True)
            s_row = jnp.where(lead_row == rank_row, rank_row, _BIG)
            gval_col = jnp.sum((s_row <= lead_col).astype(jnp.int32),
                               axis=1, keepdims=True) - 1
            enc_col = gval_col * 128 + iota_col
            finish(c, _scatter_rows(rank_col, enc_col, iota_row))


def kernel(article_sentences, article_sentences_lengths, attention, num_codes):
    B, C, S, L = _B, _C, _S, _L
    sent = article_sentences.astype(jnp.int32)
    sent_t = jnp.swapaxes(sent, 1, 2)
    lens = article_sentences_lengths.astype(jnp.int32).reshape(B, 1, S)
    nc = num_codes.astype(jnp.int32).reshape(B, 1, 1)

    out = pl.pallas_call(
        _body,
        grid=(B,),
        in_specs=[
            pl.BlockSpec((1, S, L), lambda b: (b, 0, 0)),
            pl.BlockSpec((1, L, S), lambda b: (b, 0, 0)),
            pl.BlockSpec((1, 1, S), lambda b: (b, 0, 0)),
            pl.BlockSpec((1, C, S, L), lambda b: (b, 0, 0, 0)),
            pl.BlockSpec((1, 1, 1), lambda b: (b, 0, 0)),
        ],
        out_specs=[
            pl.BlockSpec((1, C, S), lambda b: (b, 0, 0)),
            pl.BlockSpec((1, C, S), lambda b: (b, 0, 0)),
            pl.BlockSpec((1, C, S), lambda b: (b, 0, 0)),
        ],
        out_shape=[
            jax.ShapeDtypeStruct((B, C, S), jnp.float32),
            jax.ShapeDtypeStruct((B, C, S), jnp.int32),
            jax.ShapeDtypeStruct((B, C, S), jnp.int32),
        ],
        compiler_params=pltpu.CompilerParams(
            dimension_semantics=("parallel",),
        ),
    )(sent, sent_t, lens, attention, nc)
    return out[0], out[1], out[2]
